# baseline (device time: 41616 ns/iter reference)
import jax
import jax.numpy as jnp
from jax import lax
from jax.experimental import pallas as pl
from jax.experimental.pallas import tpu as pltpu

N_DEV = 4
B = 2
SQ_PER = 128
D_MODEL = 512
HQ = 16
H_PER = 4
DH = 64
DQ_PER = H_PER * DH
SKV = 128
SCALE = 0.125
NEG = -1e9


def kernel(x, Wq, K_ext, V_ext, Wo):
    def body(x_ref, wq_ref, k_ref, v_ref, wo_ref, out_ref,
             wq_all, wo_all, send_sems, recv_wq_sems, recv_wo_sems):
        my = lax.axis_index("i")

        barrier = pltpu.get_barrier_semaphore()
        for d in range(1, N_DEV):
            pl.semaphore_signal(
                barrier, inc=1,
                device_id=((my + d) % N_DEV,),
                device_id_type=pl.DeviceIdType.MESH,
            )
        pl.semaphore_wait(barrier, N_DEV - 1)

        sends = []
        for d in range(1, N_DEV):
            tgt = (my + d) % N_DEV
            for si, (src, dst_all, rsems) in enumerate(
                ((wq_ref, wq_all, recv_wq_sems),
                 (wo_ref, wo_all, recv_wo_sems))
            ):
                rdma = pltpu.make_async_remote_copy(
                    src_ref=src,
                    dst_ref=dst_all.at[my],
                    send_sem=send_sems.at[2 * (d - 1) + si],
                    recv_sem=rsems.at[my],
                    device_id=(tgt,),
                    device_id_type=pl.DeviceIdType.MESH,
                )
                rdma.start()
                sends.append(rdma)

        wq_all[pl.ds(my, 1)] = wq_ref[...][None]
        wo_all[pl.ds(my, 1)] = wo_ref[...][None]

        ri = lax.broadcasted_iota(jnp.int32, (SQ_PER, SKV), 0)
        ci = lax.broadcasted_iota(jnp.int32, (SQ_PER, SKV), 1)
        qb = my * 2 + ri // 64
        kb = ci // 64
        maskf = jnp.where((qb == kb) | ((kb % 4) == (qb % 4)), 0.0, NEG)
        rvec = lax.broadcasted_iota(jnp.int32, (SQ_PER, 1), 0)
        qbv = my * 2 + rvec // 64
        keepf = jnp.where((qbv % 4) <= 1, 1.0, 0.0)

        for blk in range(N_DEV):
            @pl.when(blk != my)
            def _(blk=blk):
                for dst_all, rsems in ((wq_all, recv_wq_sems),
                                       (wo_all, recv_wo_sems)):
                    pltpu.make_async_remote_copy(
                        src_ref=dst_all.at[blk],
                        dst_ref=dst_all.at[blk],
                        send_sem=send_sems.at[0],
                        recv_sem=rsems.at[blk],
                        device_id=(0,),
                        device_id_type=pl.DeviceIdType.MESH,
                    ).wait_recv()

            for b in range(B):
                q = jnp.dot(x_ref[b], wq_all[blk],
                            preferred_element_type=jnp.float32)
                ctxs = []
                for j in range(H_PER):
                    g = blk * H_PER + j
                    qh = q[:, j * DH:(j + 1) * DH]
                    kh = k_ref[b, :, g, :]
                    vh = v_ref[b, :, g, :]
                    s = lax.dot_general(
                        qh, kh, (((1,), (1,)), ((), ())),
                        preferred_element_type=jnp.float32) * SCALE
                    s = s + maskf
                    m = jnp.max(s, axis=-1, keepdims=True)
                    w = jnp.exp(s - m)
                    w = w / jnp.sum(w, axis=-1, keepdims=True)
                    ctxs.append(jnp.dot(w, vh,
                                        preferred_element_type=jnp.float32))
                ctx = jnp.concatenate(ctxs, axis=1) * keepf
                contrib = jnp.dot(ctx, wo_all[blk],
                                  preferred_element_type=jnp.float32)
                if blk == 0:
                    out_ref[b] = contrib
                else:
                    out_ref[b] = out_ref[b] + contrib

        for r in sends:
            r.wait_send()

    return pl.pallas_call(
        body,
        out_shape=jax.ShapeDtypeStruct((B, SQ_PER, D_MODEL), jnp.float32),
        in_specs=[pl.BlockSpec(memory_space=pltpu.VMEM)] * 5,
        out_specs=pl.BlockSpec(memory_space=pltpu.VMEM),
        scratch_shapes=[
            pltpu.VMEM((N_DEV, D_MODEL, DQ_PER), jnp.float32),
            pltpu.VMEM((N_DEV, DQ_PER, D_MODEL), jnp.float32),
            pltpu.SemaphoreType.DMA((2 * (N_DEV - 1),)),
            pltpu.SemaphoreType.DMA((N_DEV,)),
            pltpu.SemaphoreType.DMA((N_DEV,)),
        ],
        compiler_params=pltpu.CompilerParams(collective_id=0),
    )(x, Wq, K_ext, V_ext, Wo)


# device time: 31158 ns/iter; 1.3356x vs baseline; 1.3356x over previous
import jax
import jax.numpy as jnp
from jax import lax
from jax.experimental import pallas as pl
from jax.experimental.pallas import tpu as pltpu

N_DEV = 4
B = 2
SQ_PER = 128
D_MODEL = 512
HQ = 16
H_PER = 4
DH = 64
DQ_PER = H_PER * DH
SKV = 128
SCALE = 0.125
NEG = -1e9


def kernel(x, Wq, K_ext, V_ext, Wo):
    def body(x_ref, wq_ref, k_ref, v_ref, wo_ref, out_ref,
             wq_all, wo_all, send_sems, recv_wq_sems, recv_wo_sems):
        my = lax.axis_index("i")

        barrier = pltpu.get_barrier_semaphore()
        for d in range(1, N_DEV):
            pl.semaphore_signal(
                barrier, inc=1,
                device_id=((my + d) % N_DEV,),
                device_id_type=pl.DeviceIdType.MESH,
            )
        pl.semaphore_wait(barrier, N_DEV - 1)

        wq_all[pl.ds(my, 1)] = wq_ref[...].astype(jnp.bfloat16)[None]
        wo_all[pl.ds(my, 1)] = wo_ref[...].astype(jnp.bfloat16)[None]

        sends = []
        for d in range(1, N_DEV):
            tgt = (my + d) % N_DEV
            for si, (dst_all, rsems) in enumerate(
                ((wq_all, recv_wq_sems),
                 (wo_all, recv_wo_sems))
            ):
                rdma = pltpu.make_async_remote_copy(
                    src_ref=dst_all.at[my],
                    dst_ref=dst_all.at[my],
                    send_sem=send_sems.at[2 * (d - 1) + si],
                    recv_sem=rsems.at[my],
                    device_id=(tgt,),
                    device_id_type=pl.DeviceIdType.MESH,
                )
                rdma.start()
                sends.append(rdma)

        ri = lax.broadcasted_iota(jnp.int32, (SQ_PER, SKV), 0)
        ci = lax.broadcasted_iota(jnp.int32, (SQ_PER, SKV), 1)
        qb = my * 2 + ri // 64
        kb = ci // 64
        maskf = jnp.where((qb == kb) | ((kb % 4) == (qb % 4)), 0.0, NEG)
        rvec = lax.broadcasted_iota(jnp.int32, (SQ_PER, 1), 0)
        qbv = my * 2 + rvec // 64
        keepf = jnp.where((qbv % 4) <= 1, 1.0, 0.0)

        for blk in range(N_DEV):
            @pl.when(blk != my)
            def _(blk=blk):
                for dst_all, rsems in ((wq_all, recv_wq_sems),
                                       (wo_all, recv_wo_sems)):
                    pltpu.make_async_remote_copy(
                        src_ref=dst_all.at[blk],
                        dst_ref=dst_all.at[blk],
                        send_sem=send_sems.at[0],
                        recv_sem=rsems.at[blk],
                        device_id=(0,),
                        device_id_type=pl.DeviceIdType.MESH,
                    ).wait_recv()

            for b in range(B):
                xb = x_ref[b].astype(jnp.bfloat16)
                q = jnp.dot(xb, wq_all[blk],
                            preferred_element_type=jnp.float32)
                ctxs = []
                for j in range(H_PER):
                    g = blk * H_PER + j
                    qh = q[:, j * DH:(j + 1) * DH]
                    kh = k_ref[b, :, g, :].astype(jnp.bfloat16)
                    vh = v_ref[b, :, g, :].astype(jnp.bfloat16)
                    s = lax.dot_general(
                        qh.astype(jnp.bfloat16), kh,
                        (((1,), (1,)), ((), ())),
                        preferred_element_type=jnp.float32) * SCALE
                    s = s + maskf
                    m = jnp.max(s, axis=-1, keepdims=True)
                    w = jnp.exp(s - m)
                    w = w / jnp.sum(w, axis=-1, keepdims=True)
                    ctxs.append(jnp.dot(w.astype(jnp.bfloat16), vh,
                                        preferred_element_type=jnp.float32))
                ctx = jnp.concatenate(ctxs, axis=1) * keepf
                contrib = jnp.dot(ctx.astype(jnp.bfloat16), wo_all[blk],
                                  preferred_element_type=jnp.float32)
                if blk == 0:
                    out_ref[b] = contrib
                else:
                    out_ref[b] = out_ref[b] + contrib

        for r in sends:
            r.wait_send()

    return pl.pallas_call(
        body,
        out_shape=jax.ShapeDtypeStruct((B, SQ_PER, D_MODEL), jnp.float32),
        in_specs=[pl.BlockSpec(memory_space=pltpu.VMEM)] * 5,
        out_specs=pl.BlockSpec(memory_space=pltpu.VMEM),
        scratch_shapes=[
            pltpu.VMEM((N_DEV, D_MODEL, DQ_PER), jnp.bfloat16),
            pltpu.VMEM((N_DEV, DQ_PER, D_MODEL), jnp.bfloat16),
            pltpu.SemaphoreType.DMA((2 * (N_DEV - 1),)),
            pltpu.SemaphoreType.DMA((N_DEV,)),
            pltpu.SemaphoreType.DMA((N_DEV,)),
        ],
        compiler_params=pltpu.CompilerParams(collective_id=0),
    )(x, Wq, K_ext, V_ext, Wo)


# device time: 27823 ns/iter; 1.4957x vs baseline; 1.1199x over previous
import jax
import jax.numpy as jnp
from jax import lax
from jax.experimental import pallas as pl
from jax.experimental.pallas import tpu as pltpu

N_DEV = 4
B = 2
SQ_PER = 128
D_MODEL = 512
HQ = 16
H_PER = 4
DH = 64
DQ_PER = H_PER * DH
SKV = 128
SCALE = 0.125
NEG = -1e9
BSQ = B * SQ_PER


def kernel(x, Wq, K_ext, V_ext, Wo):
    def body(x_ref, wq_ref, k_ref, v_ref, wo_ref, out_ref,
             wq_all, wo_all, send_sems, recv_wq_sems, recv_wo_sems):
        my = lax.axis_index("i")
        left = (my - 1) % N_DEV
        right = (my + 1) % N_DEV

        barrier = pltpu.get_barrier_semaphore()
        for nbr in (left, right):
            pl.semaphore_signal(
                barrier, inc=1,
                device_id=(nbr,),
                device_id_type=pl.DeviceIdType.MESH,
            )
        pl.semaphore_wait(barrier, 2)

        wq_all[pl.ds(my, 1)] = wq_ref[...].astype(jnp.bfloat16)[None]
        wo_all[pl.ds(my, 1)] = wo_ref[...].astype(jnp.bfloat16)[None]

        def copy(dst_all, rsems, slot, tgt, si):
            return pltpu.make_async_remote_copy(
                src_ref=dst_all.at[slot],
                dst_ref=dst_all.at[slot],
                send_sem=send_sems.at[si],
                recv_sem=rsems.at[slot],
                device_id=(tgt,),
                device_id_type=pl.DeviceIdType.MESH,
            )

        sends = []
        for si, (dst_all, rsems, tgt) in enumerate((
            (wq_all, recv_wq_sems, left),
            (wq_all, recv_wq_sems, right),
            (wo_all, recv_wo_sems, left),
            (wo_all, recv_wo_sems, right),
        )):
            r = copy(dst_all, rsems, my, tgt, si)
            r.start()
            sends.append(r)

        fwd_wq = copy(wq_all, recv_wq_sems, left, right, 4)
        fwd_wq.wait_recv()
        fwd_wq.start()
        sends.append(fwd_wq)
        fwd_wo = copy(wo_all, recv_wo_sems, right, left, 5)
        fwd_wo.wait_recv()
        fwd_wo.start()
        sends.append(fwd_wo)

        ri = lax.broadcasted_iota(jnp.int32, (SQ_PER, SKV), 0)
        ci = lax.broadcasted_iota(jnp.int32, (SQ_PER, SKV), 1)
        qb = my * 2 + ri // 64
        kb = ci // 64
        maskf = jnp.where((qb == kb) | ((kb % 4) == (qb % 4)), 0.0, NEG)
        rvec = lax.broadcasted_iota(jnp.int32, (BSQ, 1), 0)
        qbv = my * 2 + (rvec % SQ_PER) // 64
        keepf = jnp.where((qbv % 4) <= 1, 1.0, 0.0)

        x2 = jnp.reshape(x_ref[...], (BSQ, D_MODEL)).astype(jnp.bfloat16)

        for blk in range(N_DEV):
            @pl.when(jnp.logical_and(blk != my, blk != left))
            def _(blk=blk):
                copy(wq_all, recv_wq_sems, blk, 0, 0).wait_recv()

            @pl.when(jnp.logical_and(blk != my, blk != right))
            def _(blk=blk):
                copy(wo_all, recv_wo_sems, blk, 0, 0).wait_recv()

            q = jnp.dot(x2, wq_all[blk],
                        preferred_element_type=jnp.float32)
            ctxs = []
            for b in range(B):
                row = []
                for j in range(H_PER):
                    g = blk * H_PER + j
                    qh = q[b * SQ_PER:(b + 1) * SQ_PER,
                           j * DH:(j + 1) * DH]
                    kh = k_ref[b, :, g, :].astype(jnp.bfloat16)
                    vh = v_ref[b, :, g, :].astype(jnp.bfloat16)
                    s = lax.dot_general(
                        qh.astype(jnp.bfloat16), kh,
                        (((1,), (1,)), ((), ())),
                        preferred_element_type=jnp.float32) * SCALE
                    s = s + maskf
                    m = jnp.max(s, axis=-1, keepdims=True)
                    w = jnp.exp(s - m)
                    w = w / jnp.sum(w, axis=-1, keepdims=True)
                    row.append(jnp.dot(w.astype(jnp.bfloat16), vh,
                                       preferred_element_type=jnp.float32))
                ctxs.append(jnp.concatenate(row, axis=1))
            ctx = jnp.concatenate(ctxs, axis=0) * keepf
            contrib = jnp.dot(ctx.astype(jnp.bfloat16), wo_all[blk],
                              preferred_element_type=jnp.float32)
            for b in range(B):
                piece = contrib[b * SQ_PER:(b + 1) * SQ_PER, :]
                if blk == 0:
                    out_ref[b] = piece
                else:
                    out_ref[b] = out_ref[b] + piece

        for r in sends:
            r.wait_send()

    return pl.pallas_call(
        body,
        out_shape=jax.ShapeDtypeStruct((B, SQ_PER, D_MODEL), jnp.float32),
        in_specs=[pl.BlockSpec(memory_space=pltpu.VMEM)] * 5,
        out_specs=pl.BlockSpec(memory_space=pltpu.VMEM),
        scratch_shapes=[
            pltpu.VMEM((N_DEV, D_MODEL, DQ_PER), jnp.bfloat16),
            pltpu.VMEM((N_DEV, DQ_PER, D_MODEL), jnp.bfloat16),
            pltpu.SemaphoreType.DMA((6,)),
            pltpu.SemaphoreType.DMA((N_DEV,)),
            pltpu.SemaphoreType.DMA((N_DEV,)),
        ],
        compiler_params=pltpu.CompilerParams(collective_id=0),
    )(x, Wq, K_ext, V_ext, Wo)


# device time: 21893 ns/iter; 1.9009x vs baseline; 1.2709x over previous
import jax
import jax.numpy as jnp
from jax import lax
from jax.experimental import pallas as pl
from jax.experimental.pallas import tpu as pltpu

N_DEV = 4
B = 2
SQ_PER = 128
D_MODEL = 512
HQ = 16
H_PER = 4
DH = 64
DQ_PER = H_PER * DH
SKV = 128
HSKV = H_PER * SKV
SCALE = 0.125
NEG = -1e9
BSQ = B * SQ_PER


def kernel(x, Wq, K_ext, V_ext, Wo):
    def body(x_ref, wq_ref, k_ref, v_ref, wo_ref, out_ref,
             wq_all, wo_all, send_sems, recv_wq_sems, recv_wo_sems):
        my = lax.axis_index("i")
        left = (my - 1) % N_DEV
        right = (my + 1) % N_DEV
        diag = (my + 2) % N_DEV

        barrier = pltpu.get_barrier_semaphore()
        for nbr in (left, right):
            pl.semaphore_signal(
                barrier, inc=1,
                device_id=(nbr,),
                device_id_type=pl.DeviceIdType.MESH,
            )
        pl.semaphore_wait(barrier, 2)

        wq_all[pl.ds(my, 1)] = wq_ref[...].astype(jnp.bfloat16)[None]
        wo_all[pl.ds(my, 1)] = wo_ref[...].astype(jnp.bfloat16)[None]

        def copy(dst_all, rsems, slot, tgt, si):
            return pltpu.make_async_remote_copy(
                src_ref=dst_all.at[slot],
                dst_ref=dst_all.at[slot],
                send_sem=send_sems.at[si],
                recv_sem=rsems.at[slot],
                device_id=(tgt,),
                device_id_type=pl.DeviceIdType.MESH,
            )

        sends = []
        for si, (dst_all, rsems, tgt) in enumerate((
            (wq_all, recv_wq_sems, left),
            (wo_all, recv_wo_sems, right),
            (wo_all, recv_wo_sems, left),
            (wq_all, recv_wq_sems, right),
        )):
            r = copy(dst_all, rsems, my, tgt, si)
            r.start()
            sends.append(r)

        blks = (my, right, diag, left)

        def pad_row(tile, h):
            parts = []
            if h > 0:
                parts.append(jnp.zeros((SKV, h * DH), jnp.bfloat16))
            parts.append(tile)
            if h < H_PER - 1:
                parts.append(jnp.zeros((SKV, (H_PER - 1 - h) * DH),
                                       jnp.bfloat16))
            return parts[0] if len(parts) == 1 else jnp.concatenate(
                parts, axis=1)

        K2 = []
        V2 = []
        for t in range(N_DEV):
            k2t, v2t = [], []
            for b in range(B):
                krows, vrows = [], []
                for h in range(H_PER):
                    g = blks[t] * H_PER + h
                    kh = k_ref[b, :, g, :].astype(jnp.bfloat16)
                    vh = v_ref[b, :, g, :].astype(jnp.bfloat16)
                    krows.append(pad_row(kh, h))
                    vrows.append(pad_row(vh, h))
                k2t.append(jnp.concatenate(krows, axis=0))
                v2t.append(jnp.concatenate(vrows, axis=0))
            K2.append(k2t)
            V2.append(v2t)

        fwd_wq = copy(wq_all, recv_wq_sems, left, right, 4)
        fwd_wq.wait_recv()
        fwd_wq.start()
        sends.append(fwd_wq)
        fwd_wo = copy(wo_all, recv_wo_sems, right, left, 5)
        fwd_wo.wait_recv()
        fwd_wo.start()
        sends.append(fwd_wo)

        ri = lax.broadcasted_iota(jnp.int32, (SQ_PER, SKV), 0)
        ci = lax.broadcasted_iota(jnp.int32, (SQ_PER, SKV), 1)
        qb = my * 2 + ri // 64
        kb = ci // 64
        maskf = jnp.where((qb == kb) | ((kb % 4) == (qb % 4)), 0.0, NEG)
        maskd = jnp.concatenate([maskf] * H_PER, axis=1)
        rvec = lax.broadcasted_iota(jnp.int32, (BSQ, 1), 0)
        qbv = my * 2 + (rvec % SQ_PER) // 64
        keepf = jnp.where((qbv % 4) <= 1, 1.0, 0.0)

        x2 = jnp.reshape(x_ref[...], (BSQ, D_MODEL)).astype(jnp.bfloat16)

        for t in range(N_DEV):
            blk = blks[t]
            if t == 1:
                copy(wq_all, recv_wq_sems, blk, 0, 0).wait_recv()
            elif t == 2:
                copy(wq_all, recv_wq_sems, blk, 0, 0).wait_recv()
                copy(wo_all, recv_wo_sems, blk, 0, 0).wait_recv()
            elif t == 3:
                copy(wo_all, recv_wo_sems, blk, 0, 0).wait_recv()

            q = jnp.dot(x2, wq_all[blk],
                        preferred_element_type=jnp.float32)
            q16 = q.astype(jnp.bfloat16)
            ctxs = []
            for b in range(B):
                qh = q16[b * SQ_PER:(b + 1) * SQ_PER, :]
                s = lax.dot_general(
                    qh, K2[t][b], (((1,), (1,)), ((), ())),
                    preferred_element_type=jnp.float32) * SCALE
                s = (s + maskd).reshape(SQ_PER, H_PER, SKV)
                m = jnp.max(s, axis=-1, keepdims=True)
                w = jnp.exp(s - m)
                w = w / jnp.sum(w, axis=-1, keepdims=True)
                w16 = w.reshape(SQ_PER, HSKV).astype(jnp.bfloat16)
                ctxs.append(jnp.dot(w16, V2[t][b],
                                    preferred_element_type=jnp.float32))
            ctx = jnp.concatenate(ctxs, axis=0) * keepf
            contrib = jnp.dot(ctx.astype(jnp.bfloat16), wo_all[blk],
                              preferred_element_type=jnp.float32)
            for b in range(B):
                piece = contrib[b * SQ_PER:(b + 1) * SQ_PER, :]
                if t == 0:
                    out_ref[b] = piece
                else:
                    out_ref[b] = out_ref[b] + piece

        for r in sends:
            r.wait_send()

    return pl.pallas_call(
        body,
        out_shape=jax.ShapeDtypeStruct((B, SQ_PER, D_MODEL), jnp.float32),
        in_specs=[pl.BlockSpec(memory_space=pltpu.VMEM)] * 5,
        out_specs=pl.BlockSpec(memory_space=pltpu.VMEM),
        scratch_shapes=[
            pltpu.VMEM((N_DEV, D_MODEL, DQ_PER), jnp.bfloat16),
            pltpu.VMEM((N_DEV, DQ_PER, D_MODEL), jnp.bfloat16),
            pltpu.SemaphoreType.DMA((6,)),
            pltpu.SemaphoreType.DMA((N_DEV,)),
            pltpu.SemaphoreType.DMA((N_DEV,)),
        ],
        compiler_params=pltpu.CompilerParams(collective_id=0),
    )(x, Wq, K_ext, V_ext, Wo)
